# baseline (speedup 1.0000x reference)
import functools

import jax
import jax.numpy as jnp
from jax.experimental import pallas as pl
from jax.experimental.pallas import tpu as pltpu

_HID = 32


def _rollout_kernel(T, s0_ref, w_in_ref, b_in_ref, WcT_ref, bc_ref,
                    Wx2hT_ref, bx2h_ref, Wh2hT_ref, bh2h_ref,
                    w_outT_ref, b_out_ref, out_ref, xcat_ref):
    f32 = jnp.float32
    s = s0_ref[...]
    w_in = w_in_ref[...]
    b_in = b_in_ref[...]
    WcT = WcT_ref[...]
    bc = bc_ref[...]
    Wx2hT = Wx2hT_ref[...]
    bx2h = bx2h_ref[...]
    Wh2hT = Wh2hT_ref[...]
    bh2h = bh2h_ref[...]
    w_outT = w_outT_ref[...]
    b_out = b_out_ref[...]
    mc = s.shape[1]
    cn = jnp.zeros((_HID, mc), f32)
    hn = jnp.zeros((_HID, mc), f32)
    for t in range(T):
        h_in = jnp.maximum(w_in * s + b_in, 0.0)
        xcat_ref[0:32, :] = h_in
        x_gcn = jax.lax.dot(WcT, h_in, preferred_element_type=f32) + bc
        xcat_ref[32:64, :] = jnp.maximum(x_gcn, 0.0)
        g = jax.lax.dot(Wx2hT, xcat_ref[...], preferred_element_type=f32)
        g = ((g + bx2h)
             + jax.lax.dot(Wh2hT, hn, preferred_element_type=f32)) + bh2h
        ig = jax.nn.sigmoid(g[0:32])
        fg = jax.nn.sigmoid(g[32:64])
        cg = jnp.tanh(g[64:96])
        og = jax.nn.sigmoid(g[96:128])
        cn = cn * fg + ig * cg
        hn = og * jnp.tanh(cn)
        s = jax.lax.dot(w_outT, hn, preferred_element_type=f32) + b_out
        out_ref[pl.ds(t, 1), :] = s


def kernel(x, A_q, A_h, A, W_in, b_in, Wc, bc, Wx2h, bx2h, Wh2h, bh2h,
           W_out, b_out):
    B, T, N = x.shape
    M = B * N
    f32 = jnp.float32

    s0 = x[:, 0, :].reshape(1, M).astype(f32)
    Wx2hT = Wx2h.T
    Wh2hT = Wh2h.T
    bx2h_c = bx2h.reshape(4 * _HID, 1)
    bh2h_c = bh2h.reshape(4 * _HID, 1)
    w_in = W_in.reshape(_HID, 1)
    b_in_c = b_in.reshape(_HID, 1)
    WcT = Wc.T
    bc_c = bc.reshape(_HID, 1)
    w_outT = W_out.reshape(1, _HID)
    b_out_c = b_out.reshape(1, 1)

    grid = 1
    for g in (5, 10, 25, 2, 4, 8):
        if M % g == 0 and (M // g) % 128 == 0:
            grid = g
            break
    mc = M // grid

    rep = lambda i: (0, 0)
    col = lambda i: (0, i)
    outT = pl.pallas_call(
        functools.partial(_rollout_kernel, T),
        grid=(grid,),
        in_specs=[
            pl.BlockSpec((1, mc), col),
            pl.BlockSpec((_HID, 1), rep),
            pl.BlockSpec((_HID, 1), rep),
            pl.BlockSpec((_HID, _HID), rep),
            pl.BlockSpec((_HID, 1), rep),
            pl.BlockSpec((4 * _HID, 2 * _HID), rep),
            pl.BlockSpec((4 * _HID, 1), rep),
            pl.BlockSpec((4 * _HID, _HID), rep),
            pl.BlockSpec((4 * _HID, 1), rep),
            pl.BlockSpec((1, _HID), rep),
            pl.BlockSpec((1, 1), rep),
        ],
        out_specs=pl.BlockSpec((T, mc), col),
        out_shape=jax.ShapeDtypeStruct((T, M), f32),
        scratch_shapes=[pltpu.VMEM((2 * _HID, mc), f32)],
        compiler_params=pltpu.CompilerParams(
            dimension_semantics=("arbitrary",)),
    )(s0, w_in, b_in_c, WcT, bc_c, Wx2hT, bx2h_c, Wh2hT, bh2h_c,
      w_outT, b_out_c)

    return outT.reshape(T, B, N).transpose(1, 0, 2)


# speedup vs baseline: 1.1403x; 1.1403x over previous
import functools

import jax
import jax.numpy as jnp
from jax.experimental import pallas as pl
from jax.experimental.pallas import tpu as pltpu

_HID = 32


def _rollout_kernel(T, s0_ref, w_in_ref, b_in_ref, WcT_ref, bc_ref,
                    Wx2hT_ref, bx2h_ref, Wcomb_ref, bh2h_ref,
                    b_out_ref, out_ref, xcat_ref):
    f32 = jnp.float32
    s = s0_ref[...]
    w_in = w_in_ref[...]
    b_in = b_in_ref[...]
    WcT = WcT_ref[...]
    bc = bc_ref[...]
    Wx2hT = Wx2hT_ref[...]
    bx2h = bx2h_ref[...]
    Wcomb = Wcomb_ref[...]
    bh2h = bh2h_ref[...]
    b_out = b_out_ref[...]
    mc = s.shape[1]
    cn = jnp.zeros((_HID, mc), f32)
    gh = jnp.zeros((4 * _HID, mc), f32)
    for t in range(T):
        h_in = jnp.maximum(w_in * s + b_in, 0.0)
        xcat_ref[0:32, :] = h_in
        x_gcn = jax.lax.dot(WcT, h_in, preferred_element_type=f32) + bc
        xcat_ref[32:64, :] = jnp.maximum(x_gcn, 0.0)
        g = jax.lax.dot(Wx2hT, xcat_ref[...], preferred_element_type=f32)
        g = ((g + bx2h) + gh) + bh2h
        ig = jax.nn.sigmoid(g[0:32])
        fg = jax.nn.sigmoid(g[32:64])
        cg = jnp.tanh(g[64:96])
        og = jax.nn.sigmoid(g[96:128])
        cn = cn * fg + ig * cg
        hn = og * jnp.tanh(cn)
        comb = jax.lax.dot(Wcomb, hn, preferred_element_type=f32)
        gh = comb[0:4 * _HID]
        s = comb[4 * _HID:4 * _HID + 1] + b_out
        out_ref[pl.ds(t, 1), :] = s


def kernel(x, A_q, A_h, A, W_in, b_in, Wc, bc, Wx2h, bx2h, Wh2h, bh2h,
           W_out, b_out):
    B, T, N = x.shape
    M = B * N
    f32 = jnp.float32

    s0 = x[:, 0, :].reshape(1, M).astype(f32)
    Wx2hT = Wx2h.T
    Wh2hT = Wh2h.T
    bx2h_c = bx2h.reshape(4 * _HID, 1)
    bh2h_c = bh2h.reshape(4 * _HID, 1)
    w_in = W_in.reshape(_HID, 1)
    b_in_c = b_in.reshape(_HID, 1)
    WcT = Wc.T
    bc_c = bc.reshape(_HID, 1)
    Wcomb = jnp.concatenate([
        Wh2hT, W_out.reshape(1, _HID),
        jnp.zeros((7, _HID), f32)], axis=0)
    b_out_c = b_out.reshape(1, 1)

    grid = 1
    for g in (5, 10, 25, 2, 4, 8):
        if M % g == 0 and (M // g) % 128 == 0:
            grid = g
            break
    mc = M // grid

    rep = lambda i: (0, 0)
    col = lambda i: (0, i)
    outT = pl.pallas_call(
        functools.partial(_rollout_kernel, T),
        grid=(grid,),
        in_specs=[
            pl.BlockSpec((1, mc), col),
            pl.BlockSpec((_HID, 1), rep),
            pl.BlockSpec((_HID, 1), rep),
            pl.BlockSpec((_HID, _HID), rep),
            pl.BlockSpec((_HID, 1), rep),
            pl.BlockSpec((4 * _HID, 2 * _HID), rep),
            pl.BlockSpec((4 * _HID, 1), rep),
            pl.BlockSpec((4 * _HID + 8, _HID), rep),
            pl.BlockSpec((4 * _HID, 1), rep),
            pl.BlockSpec((1, 1), rep),
        ],
        out_specs=pl.BlockSpec((T, mc), col),
        out_shape=jax.ShapeDtypeStruct((T, M), f32),
        scratch_shapes=[pltpu.VMEM((2 * _HID, mc), f32)],
        compiler_params=pltpu.CompilerParams(
            dimension_semantics=("arbitrary",)),
    )(s0, w_in, b_in_c, WcT, bc_c, Wx2hT, bx2h_c, Wcomb, bh2h_c,
      b_out_c)

    return outT.reshape(T, B, N).transpose(1, 0, 2)


# speedup vs baseline: 1.8549x; 1.6266x over previous
import functools

import jax
import jax.numpy as jnp
from jax.experimental import pallas as pl
from jax.experimental.pallas import tpu as pltpu

_HID = 32


def _rollout_kernel(T, s0_ref, w_in_ref, b_in_ref, WcT_ref, bc_ref,
                    W_allT_ref, bsum_ref, w_outT_ref, b_out_ref, out_ref,
                    xcat_ref):
    f32 = jnp.float32
    s = s0_ref[...]
    w_in = w_in_ref[...]
    b_in = b_in_ref[...]
    WcT = WcT_ref[...]
    bc = bc_ref[...]
    W_allT = W_allT_ref[...]
    bsum = bsum_ref[...]
    w_outT = w_outT_ref[...]
    b_out = b_out_ref[...]
    mc = s.shape[1]
    cn = jnp.zeros((_HID, mc), f32)
    xcat_ref[64:96, :] = cn
    for t in range(T):
        h_in = jnp.maximum(w_in * s + b_in, 0.0)
        xcat_ref[0:32, :] = h_in
        x_gcn = jax.lax.dot(WcT, h_in, preferred_element_type=f32) + bc
        xcat_ref[32:64, :] = jnp.maximum(x_gcn, 0.0)
        g = jax.lax.dot(W_allT, xcat_ref[...], preferred_element_type=f32) + bsum
        ig = 0.5 + 0.5 * jnp.tanh(g[0:32])
        fg = 0.5 + 0.5 * jnp.tanh(g[32:64])
        cg = jnp.tanh(g[64:96])
        og = 0.5 + 0.5 * jnp.tanh(g[96:128])
        cn = cn * fg + ig * cg
        hn = og * jnp.tanh(cn)
        xcat_ref[64:96, :] = hn
        s = jax.lax.dot(w_outT, hn, preferred_element_type=f32) + b_out
        out_ref[pl.ds(t, 1), :] = s


def kernel(x, A_q, A_h, A, W_in, b_in, Wc, bc, Wx2h, bx2h, Wh2h, bh2h,
           W_out, b_out):
    B, T, N = x.shape
    M = B * N
    f32 = jnp.float32

    s0 = x[:, 0, :].reshape(1, M).astype(f32)
    W_allT = jnp.concatenate([Wx2h, Wh2h], axis=0).T
    bsum = (bx2h + bh2h).reshape(4 * _HID, 1)
    gate_scale = jnp.concatenate([
        jnp.full((2 * _HID, 1), 0.5, jnp.float32),
        jnp.ones((_HID, 1), jnp.float32),
        jnp.full((_HID, 1), 0.5, jnp.float32)], axis=0)
    W_allT = W_allT * gate_scale
    bsum = bsum * gate_scale
    w_in = W_in.reshape(_HID, 1)
    b_in_c = b_in.reshape(_HID, 1)
    WcT = Wc.T
    bc_c = bc.reshape(_HID, 1)
    w_outT = W_out.reshape(1, _HID)
    b_out_c = b_out.reshape(1, 1)

    grid = 1
    for g in (5, 10, 25, 2, 4, 8):
        if M % g == 0 and (M // g) % 128 == 0:
            grid = g
            break
    mc = M // grid

    rep = lambda i: (0, 0)
    col = lambda i: (0, i)
    outT = pl.pallas_call(
        functools.partial(_rollout_kernel, T),
        grid=(grid,),
        in_specs=[
            pl.BlockSpec((1, mc), col),
            pl.BlockSpec((_HID, 1), rep),
            pl.BlockSpec((_HID, 1), rep),
            pl.BlockSpec((_HID, _HID), rep),
            pl.BlockSpec((_HID, 1), rep),
            pl.BlockSpec((4 * _HID, 3 * _HID), rep),
            pl.BlockSpec((4 * _HID, 1), rep),
            pl.BlockSpec((1, _HID), rep),
            pl.BlockSpec((1, 1), rep),
        ],
        out_specs=pl.BlockSpec((T, mc), col),
        out_shape=jax.ShapeDtypeStruct((T, M), f32),
        scratch_shapes=[pltpu.VMEM((3 * _HID, mc), f32)],
        compiler_params=pltpu.CompilerParams(
            dimension_semantics=("arbitrary",)),
    )(s0, w_in, b_in_c, WcT, bc_c, W_allT, bsum, w_outT, b_out_c)

    return outT.reshape(T, B, N).transpose(1, 0, 2)
